# baseline (speedup 1.0000x reference)
import functools

import jax
import jax.numpy as jnp
from jax import lax
from jax.experimental import pallas as pl
from jax.experimental.pallas import tpu as pltpu
from jax.experimental.pallas import tpu_sc as plsc

_EPSILON = 1e-5
_BN_EPS = 1e-5

_N_CORES = 2
_N_SUB = 16
_CH = 128
_CHS = 64

_INTERPRET = False


def _node_proj_body(h_ref, w_ref, b_ref, he_ref, uh_ref, vh_ref, pw_ref):
    out = (jnp.dot(h_ref[...], w_ref[...], preferred_element_type=jnp.float32)
           + b_ref[...])
    d = he_ref.shape[1]
    he_ref[...] = out[:, :d]
    uh_ref[...] = out[:, d:2 * d]
    vh_ref[...] = out[:, 2 * d:3 * d]
    pw_ref[...] = out[:, 3 * d:]


def _node_proj(h, w_big, b_big):
    n, d = h.shape
    d_out = w_big.shape[1]
    blk = 2000
    grid = n // blk
    one = jax.ShapeDtypeStruct((n, d), jnp.float32)
    return pl.pallas_call(
        _node_proj_body,
        grid=(grid,),
        in_specs=[
            pl.BlockSpec((blk, h.shape[1]), lambda i: (i, 0)),
            pl.BlockSpec((w_big.shape[0], d_out), lambda i: (0, 0)),
            pl.BlockSpec((1, d_out), lambda i: (0, 0)),
        ],
        out_specs=[pl.BlockSpec((blk, d), lambda i: (i, 0))] * 4,
        out_shape=[one, one, one, one],
        interpret=_INTERPRET,
    )(h, w_big, b_big)


def _edge_proj_body(e_ref, w_ref, b_ref, eemb_ref, w1e_ref):
    out = (jnp.dot(e_ref[...], w_ref[...], preferred_element_type=jnp.float32)
           + b_ref[...])
    eemb_ref[...] = out[:, :128]
    w1e_ref[...] = out[:, 128:]


def _edge_proj(e8, w_bd, b_bd):
    m8 = e8.shape[0]
    blk = 4000
    grid = m8 // blk
    one = jax.ShapeDtypeStruct((m8, 128), jnp.float32)
    return pl.pallas_call(
        _edge_proj_body,
        grid=(grid,),
        in_specs=[
            pl.BlockSpec((blk, 128), lambda i: (i, 0)),
            pl.BlockSpec((128, 256), lambda i: (0, 0)),
            pl.BlockSpec((1, 256), lambda i: (0, 0)),
        ],
        out_specs=[pl.BlockSpec((blk, 128), lambda i: (i, 0))] * 2,
        out_shape=[one, one],
        interpret=_INTERPRET,
    )(e8, w_bd, b_bd)


def _edge_update_body(m_total, k, pre_ref, eemb_ref, sp_ref, gb_ref, wbig_ref,
                      enew_ref, sig_ref):
    ssum = jnp.sum(sp_ref[:, 0, :], axis=0, keepdims=True)
    ssq = jnp.sum(sp_ref[:, 1, :], axis=0, keepdims=True)
    mean = ssum * (1.0 / m_total)
    var = ssq * (1.0 / m_total) - mean * mean
    inv = jax.lax.rsqrt(var + _BN_EPS) * gb_ref[0:1, :]
    shift = gb_ref[1:2, :] - mean * inv
    inv_t = jnp.concatenate([inv] * 8, axis=1)
    shift_t = jnp.concatenate([shift] * 8, axis=1)
    pre = pre_ref[...]
    bn = pre * inv_t + shift_t
    e_new = eemb_ref[...] + jnp.maximum(bn, 0.0)
    enew_ref[...] = e_new
    s = jax.nn.sigmoid(e_new)
    sig = jnp.dot(s, wbig_ref[...], preferred_element_type=jnp.float32)
    half = sig.shape[1] // 2
    sig_ref[0, :, :] = sig[:, :half]
    sig_ref[1, :, :] = sig[:, half:]


def _edge_update(pre8, e_emb8, stats_partial, gamma, beta, w_big, k):
    m8 = pre8.shape[0]
    nw = stats_partial.shape[0]
    gb = jnp.stack([gamma, beta], axis=0)
    blk = 1000
    grid = m8 // blk
    dw = w_big.shape[1]
    return pl.pallas_call(
        functools.partial(_edge_update_body, float(m8 * 128 // k), k),
        grid=(grid,),
        in_specs=[
            pl.BlockSpec((blk, 128), lambda i: (i, 0)),
            pl.BlockSpec((blk, 128), lambda i: (i, 0)),
            pl.BlockSpec((nw, 8, k), lambda i: (0, 0, 0)),
            pl.BlockSpec((2, k), lambda i: (0, 0)),
            pl.BlockSpec((128, dw), lambda i: (0, 0)),
        ],
        out_specs=[
            pl.BlockSpec((blk, 128), lambda i: (i, 0)),
            pl.BlockSpec((2, blk, dw // 2), lambda i: (0, i, 0)),
        ],
        out_shape=[
            jax.ShapeDtypeStruct((m8, 128), jnp.float32),
            jax.ShapeDtypeStruct((2, m8, dw // 2), jnp.float32),
        ],
        interpret=_INTERPRET,
    )(pre8, e_emb8, stats_partial, gb, w_big)


def _sc_pre(pw, w1e8, edge_index, k):
    n_e = w1e8.shape[0] * w1e8.shape[1] // k
    rows_per_chunk = _CH * k // 128
    n_chunks = n_e // _CH
    n_workers = _N_CORES * _N_SUB
    chunks_per_worker = -(-n_chunks // n_workers)
    mesh = plsc.VectorSubcoreMesh(core_axis_name="c", subcore_axis_name="s",
                                  num_cores=_N_CORES, num_subcores=_N_SUB)

    @functools.partial(
        pl.kernel,
        out_type=[jax.ShapeDtypeStruct(w1e8.shape, jnp.float32),
                  jax.ShapeDtypeStruct((n_workers, 8, k), jnp.float32)],
        mesh=mesh,
        scratch_types=[
            [pltpu.VMEM((_CH,), jnp.int32)] * 2,
            [pltpu.VMEM((_CH,), jnp.int32)] * 2,
            [pltpu.VMEM((rows_per_chunk, 128), jnp.float32)] * 2,
            [pltpu.VMEM((_CH, 128), jnp.float32)] * 2,
            [pltpu.VMEM((_CH, 128), jnp.float32)] * 2,
            [pltpu.VMEM((rows_per_chunk, 128), jnp.float32)] * 2,
            pltpu.VMEM((2, k), jnp.float32),
            [pltpu.SemaphoreType.DMA] * 2,
            [pltpu.SemaphoreType.DMA] * 2,
            [pltpu.SemaphoreType.DMA] * 2,
        ],
    )
    def launch(pw_h, w1e_h, ei_h, pre_out, stats_out,
               src_b, dst_b, w1e_b, sb, db, pre_b, st_buf, sem_in, sem_g, sem_o):
        c = lax.axis_index("c")
        s = lax.axis_index("s")
        wid = c * _N_SUB + s
        zero = jnp.zeros((k,), jnp.float32)
        st_buf[0, :] = zero
        st_buf[1, :] = zero

        def fire_in(i, b):
            j = i * n_workers + wid

            @pl.when(j < n_chunks)
            def _():
                base = j * _CH
                pltpu.async_copy(ei_h.at[0, pl.ds(base, _CH)], src_b[b], sem_in[b])
                pltpu.async_copy(ei_h.at[1, pl.ds(base, _CH)], dst_b[b], sem_in[b])
                pltpu.async_copy(w1e_h.at[pl.ds(j * rows_per_chunk, rows_per_chunk)],
                                 w1e_b[b], sem_in[b])

        def wait_in(b):
            pltpu.make_async_copy(ei_h.at[0, pl.ds(0, _CH)], src_b[b], sem_in[b]).wait()
            pltpu.make_async_copy(ei_h.at[1, pl.ds(0, _CH)], dst_b[b], sem_in[b]).wait()
            pltpu.make_async_copy(w1e_h.at[pl.ds(0, rows_per_chunk)], w1e_b[b],
                                 sem_in[b]).wait()

        fire_in(0, 0)
        fire_in(1, 1)
        n_pairs = -(-chunks_per_worker // 2)

        def pair_body(p, carry):
            i0 = 2 * p
            for b in range(2):
                j = (i0 + b) * n_workers + wid

                @pl.when(j < n_chunks)
                def _():
                    wait_in(b)
                    pltpu.async_copy(pw_h.at[src_b[b]], sb[b], sem_g[b])
                    pltpu.async_copy(pw_h.at[dst_b[b]], db[b], sem_g[b])

            for b in range(2):
                i = i0 + b
                j = i * n_workers + wid
                base = j * _CH

                @pl.when(j < n_chunks)
                def _(b=b, i=i, j=j):
                    pltpu.make_async_copy(pw_h.at[src_b[b]], sb[b], sem_g[b]).wait()
                    pltpu.make_async_copy(pw_h.at[dst_b[b]], db[b], sem_g[b]).wait()

                    @pl.when(i >= 2)
                    def _():
                        pltpu.make_async_copy(pre_b[b],
                                              pre_out.at[pl.ds(0, rows_per_chunk)],
                                              sem_o[b]).wait()

                    def row_body(r, rc):
                        sm, sq = rc
                        pr = r // 8
                        pc = (r % 8) * k
                        v = (w1e_b[b][pr, pl.ds(pc, k)] + sb[b][r, pl.ds(0, k)]
                             + db[b][r, pl.ds(k, k)])
                        pre_b[b][pr, pl.ds(pc, k)] = v
                        return (sm + v, sq + v * v)

                    sm2, sq2 = lax.fori_loop(0, _CH, row_body,
                                             (st_buf[0, :], st_buf[1, :]))
                    st_buf[0, :] = sm2
                    st_buf[1, :] = sq2
                    pltpu.async_copy(pre_b[b],
                                     pre_out.at[pl.ds(j * rows_per_chunk,
                                                      rows_per_chunk)], sem_o[b])

                fire_in(i + 2, b)

            return carry

        lax.fori_loop(0, n_pairs, pair_body, 0)
        for b in range(2):
            pltpu.make_async_copy(pre_b[b], pre_out.at[pl.ds(0, rows_per_chunk)],
                                  sem_o[b]).wait()
        pltpu.sync_copy(st_buf, stats_out.at[wid, pl.ds(0, 2)])

    return launch(pw, w1e8, edge_index)


def _sc_scatter(sigma2, vh, edge_index, zeros_half):
    n_e = sigma2.shape[1]
    n = vh.shape[0]
    half = vh.shape[1] // 2
    n_chunks = n_e // _CHS
    chunks_per_tile = -(-n_chunks // _N_SUB)
    rows_per_tile = (n // _N_SUB) // 8 * 8
    rows_rem = n - rows_per_tile * _N_SUB
    mesh = plsc.VectorSubcoreMesh(core_axis_name="c", subcore_axis_name="s",
                                  num_cores=_N_CORES, num_subcores=_N_SUB)

    @functools.partial(
        pl.kernel,
        out_type=jax.ShapeDtypeStruct((_N_CORES, n, 2 * half), jnp.float32),
        mesh=mesh,
        scratch_types=[
            pltpu.VMEM_SHARED((n, 2 * half), jnp.float32),
            [pltpu.VMEM((_CHS,), jnp.int32)] * 2,
            [pltpu.VMEM((_CHS,), jnp.int32)] * 2,
            [pltpu.VMEM((_CHS, half), jnp.float32)] * 2,
            [pltpu.VMEM((_CHS, 2 * half), jnp.float32)] * 2,
            [pltpu.VMEM((_CHS, 2 * half), jnp.float32)] * 2,
            [pltpu.SemaphoreType.DMA] * 2,
            [pltpu.SemaphoreType.DMA] * 2,
        ],
    )
    def launch(sig_h, vh_h, ei_h, z_h, acc_out,
               acc, src_b, dst_b, sig_b, vh_b, comb_b, sem_in, sem_g):
        c = lax.axis_index("c")
        s = lax.axis_index("s")
        row0 = s * rows_per_tile
        pltpu.sync_copy(z_h.at[pl.ds(row0, rows_per_tile)],
                        acc.at[pl.ds(row0, rows_per_tile)])
        if rows_rem:
            tail = rows_per_tile * _N_SUB

            @pl.when(s == 0)
            def _():
                pltpu.sync_copy(z_h.at[pl.ds(tail, rows_rem)],
                                acc.at[pl.ds(tail, rows_rem)])
        plsc.subcore_barrier()

        col0 = c * half

        def fire_in(i, b):
            j = i * _N_SUB + s

            @pl.when(j < n_chunks)
            def _():
                base = j * _CHS
                pltpu.async_copy(ei_h.at[0, pl.ds(base, _CHS)], src_b[b], sem_in[b])
                pltpu.async_copy(ei_h.at[1, pl.ds(base, _CHS)], dst_b[b], sem_in[b])
                pltpu.async_copy(sig_h.at[c, pl.ds(base, _CHS)], sig_b[b], sem_in[b])

        def wait_in(b):
            pltpu.make_async_copy(ei_h.at[0, pl.ds(0, _CHS)], src_b[b], sem_in[b]).wait()
            pltpu.make_async_copy(ei_h.at[1, pl.ds(0, _CHS)], dst_b[b], sem_in[b]).wait()
            pltpu.make_async_copy(sig_h.at[c, pl.ds(0, _CHS)], sig_b[b], sem_in[b]).wait()

        def compute(b):
            def row_body(r, rc):
                for q in range(half // 16):
                    sl = pl.ds(q * 16, 16)
                    vsl = pl.ds(col0 + q * 16, 16)
                    sv = sig_b[b][r, sl]
                    comb_b[b][r, sl] = vh_b[b][r, vsl] * sv
                    comb_b[b][r, pl.ds(half + q * 16, 16)] = sv
                return rc

            lax.fori_loop(0, _CHS, row_body, 0)

        fire_in(0, 0)
        fire_in(1, 1)
        n_pairs = -(-chunks_per_tile // 2)

        def pair_body(p, carry):
            i0 = 2 * p
            for b in range(2):
                i = i0 + b
                j = i * _N_SUB + s

                @pl.when(j < n_chunks)
                def _():
                    wait_in(b)
                    pltpu.async_copy(vh_h.at[src_b[b]], vh_b[b], sem_g[b])

            for b in range(2):
                i = i0 + b
                j = i * _N_SUB + s

                @pl.when(j < n_chunks)
                def _():
                    pltpu.make_async_copy(vh_h.at[src_b[b]], vh_b[b], sem_g[b]).wait()
                    compute(b)
                    pltpu.sync_copy(comb_b[b], acc.at[dst_b[b]], add=True)

                fire_in(i + 2, b)

            return carry

        lax.fori_loop(0, n_pairs, pair_body, 0)
        plsc.subcore_barrier()
        pltpu.sync_copy(acc.at[pl.ds(row0, rows_per_tile)],
                        acc_out.at[c, pl.ds(row0, rows_per_tile)])
        if rows_rem:
            tail = rows_per_tile * _N_SUB

            @pl.when(s == 0)
            def _():
                pltpu.sync_copy(acc.at[pl.ds(tail, rows_rem)],
                                acc_out.at[c, pl.ds(tail, rows_rem)])

    return launch(sigma2, vh, edge_index, zeros_half)


def _node_update_body(hemb_ref, uh_ref, acc_ref, gb_ref, out_ref):
    half = acc_ref.shape[2] // 2
    num = jnp.concatenate([acc_ref[0, :, :half], acc_ref[1, :, :half]], axis=1)
    den = jnp.concatenate([acc_ref[0, :, half:], acc_ref[1, :, half:]], axis=1)
    x = uh_ref[...] + num / (den + _EPSILON)
    n = x.shape[0]
    mean = jnp.sum(x, axis=0, keepdims=True) / n
    var = jnp.sum((x - mean) ** 2, axis=0, keepdims=True) / n
    bn = (x - mean) * jax.lax.rsqrt(var + _BN_EPS) * gb_ref[0:1, :] + gb_ref[1:2, :]
    out_ref[...] = hemb_ref[...] + jnp.maximum(bn, 0.0)


def _node_update(h_emb, uh, acc2, gamma, beta):
    n, d = h_emb.shape
    gb = jnp.stack([gamma, beta], axis=0)
    return pl.pallas_call(
        _node_update_body,
        in_specs=[
            pl.BlockSpec((n, d), lambda: (0, 0)),
            pl.BlockSpec((n, d), lambda: (0, 0)),
            pl.BlockSpec((2, n, d), lambda: (0, 0, 0)),
            pl.BlockSpec((2, d), lambda: (0, 0)),
        ],
        out_specs=pl.BlockSpec((n, d), lambda: (0, 0)),
        out_shape=jax.ShapeDtypeStruct((n, d), jnp.float32),
        interpret=_INTERPRET,
    )(h_emb, uh, acc2, gb)


def kernel(h, e, edge_index, Wn, We, Weta, Uw, Ub, Vw, Vb, W1w, W1b, W2w, W2b,
           W3w, W3b, hbn_gamma, hbn_beta, ebn_gamma, ebn_beta):
    src = edge_index[0]
    dst = edge_index[1]
    n, d = h.shape
    m, k = e.shape

    pad = d - 2 * k
    w_node = jnp.concatenate(
        [Wn, Uw, Vw, W2w, W3w, jnp.zeros((d, pad), jnp.float32)], axis=1)
    b_node = jnp.concatenate(
        [jnp.zeros((d,), jnp.float32), Ub, Vb, W2b, W3b,
         jnp.zeros((pad,), jnp.float32)], axis=0)[None, :]
    h_emb, uh, vh, pw = _node_proj(h, w_node, b_node)

    g = 128 // k
    eye_g = jnp.eye(g, dtype=jnp.float32)
    bd_we = jnp.einsum('ab,ij->aibj', eye_g, We).reshape(128, 128)
    bd_w1 = jnp.einsum('ab,ij->aibj', eye_g, W1w).reshape(128, 128)
    w_bd = jnp.concatenate([bd_we, bd_w1], axis=1)
    b_bd = jnp.concatenate(
        [jnp.zeros((128,), jnp.float32), jnp.tile(W1b, g)], axis=0)[None, :]
    e8 = e.reshape(m // g, 128)
    e_emb8, w1e8 = _edge_proj(e8, w_bd, b_bd)

    pre8, stats_partial = _sc_pre(pw, w1e8, edge_index, k)

    half = d // 2
    w_big = jnp.zeros((128, 1024), jnp.float32)
    for mm in range(g):
        w_big = w_big.at[mm * k:(mm + 1) * k,
                         mm * half:(mm + 1) * half].set(Weta[:, :half])
        w_big = w_big.at[mm * k:(mm + 1) * k,
                         512 + mm * half:512 + (mm + 1) * half].set(Weta[:, half:])

    e_new8, sigma2p = _edge_update(pre8, e_emb8, stats_partial,
                                   ebn_gamma, ebn_beta, w_big, k)
    e_new = e_new8.reshape(m, k)
    sigma2 = sigma2p.reshape(2, m, half)

    zeros_full = jnp.zeros((n, d), jnp.float32)
    acc2 = _sc_scatter(sigma2, vh, edge_index, zeros_full)

    h_new = _node_update(h_emb, uh, acc2, hbn_gamma, hbn_beta)
    return (h_new, e_new)
